# baseline (speedup 1.0000x reference)
import jax
import jax.numpy as jnp
from jax.experimental import pallas as pl
from jax.experimental.pallas import tpu as pltpu

_GAMMA = 2.0
_ALPHA = 0.25
_LOSS_WEIGHT = 100.0
_IGNORE_INDEX = 255


def _focal_body(H, W, D, Hb, pred_ref, tgt_ref, loss_ref, cnt_ref):
    b = pl.program_id(0)
    h = pl.program_id(1)

    @pl.when((b == 0) & (h == 0))
    def _init():
        loss_ref[0, 0] = 0.0
        cnt_ref[0, 0] = 0.0

    x = pred_ref[0]
    tgt = tgt_ref[0]
    C = x.shape[0]
    WD = x.shape[2]

    cls = jax.lax.broadcasted_iota(jnp.int32, (C, Hb, WD), 0)
    oh = tgt[None, :, :] == cls

    e = jnp.exp(-jnp.abs(x))
    softplus = jnp.maximum(x, 0.0) + jnp.log1p(e)
    bce = softplus - jnp.where(oh, x, 0.0)
    inv = 1.0 / (1.0 + e)
    prob = jnp.where(x >= 0.0, inv, e * inv)
    one_m_pt = jnp.where(oh, 1.0 - prob, prob)
    af = jnp.where(oh, _ALPHA, 1.0 - _ALPHA)
    loss = bce * af * (one_m_pt * one_m_pt)
    loss_v = jnp.sum(loss, axis=0)

    row = jax.lax.broadcasted_iota(jnp.int32, (Hb, WD), 0) + h * Hb
    col = jax.lax.broadcasted_iota(jnp.int32, (Hb, WD), 1)
    yy = row.astype(jnp.float32) - (H / 2.0)
    xx = (col // D).astype(jnp.float32) - (W / 2.0)
    c_max = jnp.sqrt(jnp.float32((H / 2.0) ** 2 + (W / 2.0) ** 2))
    cmap = jnp.sqrt(yy * yy + xx * xx) / c_max + 1.0

    vis = (tgt != _IGNORE_INDEX).astype(jnp.float32)
    w_eff = cmap * vis

    loss_ref[0, 0] += jnp.sum(loss_v * w_eff)
    cnt_ref[0, 0] += jnp.sum(vis)


def kernel(pred, target):
    B, C, H, W, D = pred.shape
    WD = W * D
    Hb = 8
    pred4 = pred.reshape(B, C, H, WD)
    tgt3 = target.reshape(B, H, WD)

    import functools
    body = functools.partial(_focal_body, H, W, D, Hb)
    loss_sum, cnt = pl.pallas_call(
        body,
        grid=(B, H // Hb),
        in_specs=[
            pl.BlockSpec((1, C, Hb, WD), lambda b, h: (b, 0, h, 0)),
            pl.BlockSpec((1, Hb, WD), lambda b, h: (b, h, 0)),
        ],
        out_specs=[
            pl.BlockSpec((1, 1), lambda b, h: (0, 0), memory_space=pltpu.SMEM),
            pl.BlockSpec((1, 1), lambda b, h: (0, 0), memory_space=pltpu.SMEM),
        ],
        out_shape=[
            jax.ShapeDtypeStruct((1, 1), jnp.float32),
            jax.ShapeDtypeStruct((1, 1), jnp.float32),
        ],
    )(pred4, tgt3)
    return _LOSS_WEIGHT * loss_sum[0, 0] / cnt[0, 0]


# speedup vs baseline: 1.1164x; 1.1164x over previous
import functools

import jax
import jax.numpy as jnp
from jax.experimental import pallas as pl
from jax.experimental.pallas import tpu as pltpu

_ALPHA = 0.25
_LOSS_WEIGHT = 100.0
_IGNORE_INDEX = 255

_LOG2E = 1.4426950408889634
_LN2 = 0.6931471805599453


def _focal_body(C, Hb, WD, WCH, pred_ref, tgt_ref, w_ref, loss_ref, cnt_ref):
    b = pl.program_id(0)
    h = pl.program_id(1)

    @pl.when((b == 0) & (h == 0))
    def _init():
        loss_ref[0, 0] = 0.0
        cnt_ref[0, 0] = 0.0

    total = jnp.float32(0.0)
    cnt = jnp.float32(0.0)
    for j in range(WD // WCH):
        sl = slice(j * WCH, (j + 1) * WCH)
        tgt_j = tgt_ref[0, :, sl]
        acc = jnp.zeros((Hb, WCH), jnp.float32)
        for c in range(C):
            x = pred_ref[0, c, :, sl]
            e = jnp.exp2(jnp.abs(x) * (-_LOG2E))
            t = 1.0 + e
            sp = jnp.maximum(x, 0.0) + jnp.log2(t) * _LN2
            inv = 1.0 / t
            sig = jnp.where(x >= 0.0, inv, e * inv)
            oh = tgt_j == c
            bce = sp - jnp.where(oh, x, 0.0)
            m = jnp.where(oh, 1.0 - sig, sig)
            af = jnp.where(oh, _ALPHA, 1.0 - _ALPHA)
            acc = acc + bce * (af * (m * m))
        vis = tgt_j != _IGNORE_INDEX
        w_eff = jnp.where(vis, w_ref[:, sl], 0.0)
        total = total + jnp.sum(acc * w_eff)
        cnt = cnt + jnp.sum(jnp.where(vis, 1.0, 0.0))

    loss_ref[0, 0] += total
    cnt_ref[0, 0] += cnt


def kernel(pred, target):
    B, C, H, W, D = pred.shape
    WD = W * D
    Hb = 8
    WCH = 640
    pred4 = pred.reshape(B, C, H, WD)
    tgt3 = target.reshape(B, H, WD)

    yy = jnp.arange(H, dtype=jnp.float32) - H / 2.0
    xx = jnp.arange(W, dtype=jnp.float32) - W / 2.0
    gy, gx = jnp.meshgrid(yy, xx, indexing="ij")
    cmap = jnp.sqrt(gy * gy + gx * gx)
    cmap = cmap / jnp.maximum(cmap.max(), 1e-12) + 1.0
    wmap = jnp.repeat(cmap, D, axis=1)

    body = functools.partial(_focal_body, C, Hb, WD, WCH)
    loss_sum, cnt = pl.pallas_call(
        body,
        grid=(B, H // Hb),
        in_specs=[
            pl.BlockSpec((1, C, Hb, WD), lambda b, h: (b, 0, h, 0)),
            pl.BlockSpec((1, Hb, WD), lambda b, h: (b, h, 0)),
            pl.BlockSpec((Hb, WD), lambda b, h: (h, 0)),
        ],
        out_specs=[
            pl.BlockSpec((1, 1), lambda b, h: (0, 0), memory_space=pltpu.SMEM),
            pl.BlockSpec((1, 1), lambda b, h: (0, 0), memory_space=pltpu.SMEM),
        ],
        out_shape=[
            jax.ShapeDtypeStruct((1, 1), jnp.float32),
            jax.ShapeDtypeStruct((1, 1), jnp.float32),
        ],
    )(pred4, tgt3, wmap)
    return _LOSS_WEIGHT * loss_sum[0, 0] / cnt[0, 0]


# speedup vs baseline: 1.1926x; 1.0683x over previous
import functools

import jax
import jax.numpy as jnp
from jax.experimental import pallas as pl
from jax.experimental.pallas import tpu as pltpu

_ALPHA = 0.25
_LOSS_WEIGHT = 100.0
_IGNORE_INDEX = 255

_LOG2E = 1.4426950408889634
_LN2 = 0.6931471805599453


def _focal_body(C, Rb, RCH, pred_ref, tgt_ref, w_ref, loss_ref, cnt_ref):
    b = pl.program_id(0)
    g = pl.program_id(1)

    @pl.when((b == 0) & (g == 0))
    def _init():
        loss_ref[0, 0] = 0.0
        cnt_ref[0, 0] = 0.0

    vacc = jnp.zeros((RCH, 128), jnp.float32)
    vcnt = jnp.zeros((RCH, 128), jnp.float32)
    for j in range(Rb // RCH):
        sl = slice(j * RCH, (j + 1) * RCH)
        tgt_j = tgt_ref[0, sl, :]
        acc = jnp.zeros((RCH, 128), jnp.float32)
        for c in range(C):
            x = pred_ref[0, c, sl, :]
            e = jnp.exp2(jnp.abs(x) * (-_LOG2E))
            t = 1.0 + e
            sp = jnp.maximum(x, 0.0) + jnp.log2(t) * _LN2
            inv = 1.0 / t
            sig = jnp.where(x >= 0.0, inv, e * inv)
            oh = tgt_j == c
            bce = sp - jnp.where(oh, x, 0.0)
            m = jnp.where(oh, 1.0 - sig, sig)
            af = jnp.where(oh, _ALPHA, 1.0 - _ALPHA)
            acc = acc + bce * (af * (m * m))
        vis = tgt_j != _IGNORE_INDEX
        vacc = vacc + acc * jnp.where(vis, w_ref[sl, :], 0.0)
        vcnt = vcnt + jnp.where(vis, 1.0, 0.0)

    loss_ref[0, 0] += jnp.sum(vacc)
    cnt_ref[0, 0] += jnp.sum(vcnt)


def kernel(pred, target):
    B, C, H, W, D = pred.shape
    R = H * W * D // 128
    G = 25
    Rb = R // G
    RCH = 40

    pred3 = pred.reshape(B, C, R, 128)
    tgt2 = target.reshape(B, R, 128)

    yy = jnp.arange(H, dtype=jnp.float32) - H / 2.0
    xx = jnp.arange(W, dtype=jnp.float32) - W / 2.0
    gy, gx = jnp.meshgrid(yy, xx, indexing="ij")
    cmap = jnp.sqrt(gy * gy + gx * gx)
    cmap = cmap / jnp.maximum(cmap.max(), 1e-12) + 1.0
    wmap = jnp.broadcast_to(cmap[:, :, None], (H, W, D)).reshape(R, 128)

    body = functools.partial(_focal_body, C, Rb, RCH)
    loss_sum, cnt = pl.pallas_call(
        body,
        grid=(B, G),
        in_specs=[
            pl.BlockSpec((1, C, Rb, 128), lambda b, g: (b, 0, g, 0)),
            pl.BlockSpec((1, Rb, 128), lambda b, g: (b, g, 0)),
            pl.BlockSpec((Rb, 128), lambda b, g: (g, 0)),
        ],
        out_specs=[
            pl.BlockSpec((1, 1), lambda b, g: (0, 0), memory_space=pltpu.SMEM),
            pl.BlockSpec((1, 1), lambda b, g: (0, 0), memory_space=pltpu.SMEM),
        ],
        out_shape=[
            jax.ShapeDtypeStruct((1, 1), jnp.float32),
            jax.ShapeDtypeStruct((1, 1), jnp.float32),
        ],
    )(pred3, tgt2, wmap)
    return _LOSS_WEIGHT * loss_sum[0, 0] / cnt[0, 0]


# speedup vs baseline: 7.8626x; 6.5930x over previous
import functools

import jax
import jax.numpy as jnp
from jax.experimental import pallas as pl
from jax.experimental.pallas import tpu as pltpu

_ALPHA = 0.25
_LOSS_WEIGHT = 100.0
_IGNORE_INDEX = 255

_LOG2E = 1.4426950408889634
_LN2 = 0.6931471805599453


def _focal_body(C, Hb, D, W, pred_ref, tgt_ref, w_ref, loss_ref, cnt_ref):
    b = pl.program_id(0)
    g = pl.program_id(1)

    @pl.when((b == 0) & (g == 0))
    def _init():
        loss_ref[0, 0] = 0.0
        cnt_ref[0, 0] = 0.0

    vacc = jnp.zeros((D, W), jnp.float32)
    vcnt = jnp.zeros((D, W), jnp.float32)
    for hh in range(Hb):
        tgt_h = tgt_ref[0, hh]
        acc = jnp.zeros((D, W), jnp.float32)
        for c in range(C):
            x = pred_ref[0, c, hh]
            e = jnp.exp2(jnp.abs(x) * (-_LOG2E))
            t = 1.0 + e
            sp = jnp.maximum(x, 0.0) + jnp.log2(t) * _LN2
            inv = 1.0 / t
            sig = jnp.where(x >= 0.0, inv, e * inv)
            oh = tgt_h == c
            bce = sp - jnp.where(oh, x, 0.0)
            m = jnp.where(oh, 1.0 - sig, sig)
            af = jnp.where(oh, _ALPHA, 1.0 - _ALPHA)
            acc = acc + bce * (af * (m * m))
        vis = tgt_h != _IGNORE_INDEX
        vacc = vacc + acc * jnp.where(vis, w_ref[hh], 0.0)
        vcnt = vcnt + jnp.where(vis, 1.0, 0.0)

    loss_ref[0, 0] += jnp.sum(vacc)
    cnt_ref[0, 0] += jnp.sum(vcnt)


def kernel(pred, target):
    B, C, H, W, D = pred.shape
    Hb = 8

    predT = jnp.transpose(pred, (0, 1, 2, 4, 3))
    tgtT = jnp.transpose(target, (0, 1, 3, 2))

    yy = jnp.arange(H, dtype=jnp.float32) - H / 2.0
    xx = jnp.arange(W, dtype=jnp.float32) - W / 2.0
    gy, gx = jnp.meshgrid(yy, xx, indexing="ij")
    cmap = jnp.sqrt(gy * gy + gx * gx)
    cmap = cmap / jnp.maximum(cmap.max(), 1e-12) + 1.0
    wmapT = jnp.broadcast_to(cmap[:, None, :], (H, D, W))

    body = functools.partial(_focal_body, C, Hb, D, W)
    loss_sum, cnt = pl.pallas_call(
        body,
        grid=(B, H // Hb),
        in_specs=[
            pl.BlockSpec((1, C, Hb, D, W), lambda b, g: (b, 0, g, 0, 0)),
            pl.BlockSpec((1, Hb, D, W), lambda b, g: (b, g, 0, 0)),
            pl.BlockSpec((Hb, D, W), lambda b, g: (g, 0, 0)),
        ],
        out_specs=[
            pl.BlockSpec((1, 1), lambda b, g: (0, 0), memory_space=pltpu.SMEM),
            pl.BlockSpec((1, 1), lambda b, g: (0, 0), memory_space=pltpu.SMEM),
        ],
        out_shape=[
            jax.ShapeDtypeStruct((1, 1), jnp.float32),
            jax.ShapeDtypeStruct((1, 1), jnp.float32),
        ],
    )(predT, tgtT, wmapT)
    return _LOSS_WEIGHT * loss_sum[0, 0] / cnt[0, 0]


# speedup vs baseline: 8.9145x; 1.1338x over previous
import functools

import jax
import jax.numpy as jnp
from jax.experimental import pallas as pl
from jax.experimental.pallas import tpu as pltpu

_ALPHA = 0.25
_LOSS_WEIGHT = 100.0
_IGNORE_INDEX = 255

_LOG2E = 1.4426950408889634
_LN2 = 0.6931471805599453


def _focal_body(C, Hb, D, W, pred_ref, tgt_ref, w_ref, loss_ref, cnt_ref):
    b = pl.program_id(0)
    g = pl.program_id(1)

    @pl.when((b == 0) & (g == 0))
    def _init():
        loss_ref[0, 0] = 0.0
        cnt_ref[0, 0] = 0.0

    vacc = jnp.zeros((D, W), jnp.float32)
    vcnt = jnp.zeros((D, W), jnp.float32)
    for hh in range(Hb):
        tgt_h = tgt_ref[0, hh]
        acc = jnp.zeros((D, W), jnp.float32)
        xt = jnp.zeros((D, W), jnp.float32)
        for c in range(C):
            x = pred_ref[0, c, hh]
            e = jnp.exp2(jnp.abs(x) * (-_LOG2E))
            t = 1.0 + e
            sp = jnp.maximum(x, 0.0) + jnp.log2(t) * _LN2
            inv = 1.0 / t
            sig = jnp.where(x >= 0.0, inv, e * inv)
            acc = acc + sp * (sig * sig)
            xt = jnp.where(tgt_h == c, x, xt)
        e = jnp.exp2(jnp.abs(xt) * (-_LOG2E))
        t = 1.0 + e
        sp = jnp.maximum(xt, 0.0) + jnp.log2(t) * _LN2
        inv = 1.0 / t
        sig = jnp.where(xt >= 0.0, inv, e * inv)
        oms = 1.0 - sig
        l1 = (sp - xt) * (_ALPHA * (oms * oms))
        l0t = sp * ((1.0 - _ALPHA) * (sig * sig))
        voxel = (1.0 - _ALPHA) * acc + (l1 - l0t)
        vis = tgt_h != _IGNORE_INDEX
        vacc = vacc + voxel * jnp.where(vis, w_ref[hh], 0.0)
        vcnt = vcnt + jnp.where(vis, 1.0, 0.0)

    loss_ref[0, 0] += jnp.sum(vacc)
    cnt_ref[0, 0] += jnp.sum(vcnt)


def kernel(pred, target):
    B, C, H, W, D = pred.shape
    Hb = 8

    predT = jnp.transpose(pred, (0, 1, 2, 4, 3))
    tgtT = jnp.transpose(target, (0, 1, 3, 2))

    yy = jnp.arange(H, dtype=jnp.float32) - H / 2.0
    xx = jnp.arange(W, dtype=jnp.float32) - W / 2.0
    gy, gx = jnp.meshgrid(yy, xx, indexing="ij")
    cmap = jnp.sqrt(gy * gy + gx * gx)
    cmap = cmap / jnp.maximum(cmap.max(), 1e-12) + 1.0
    wmapT = jnp.broadcast_to(cmap[:, None, :], (H, D, W))

    body = functools.partial(_focal_body, C, Hb, D, W)
    loss_sum, cnt = pl.pallas_call(
        body,
        grid=(B, H // Hb),
        in_specs=[
            pl.BlockSpec((1, C, Hb, D, W), lambda b, g: (b, 0, g, 0, 0)),
            pl.BlockSpec((1, Hb, D, W), lambda b, g: (b, g, 0, 0)),
            pl.BlockSpec((Hb, D, W), lambda b, g: (g, 0, 0)),
        ],
        out_specs=[
            pl.BlockSpec((1, 1), lambda b, g: (0, 0), memory_space=pltpu.SMEM),
            pl.BlockSpec((1, 1), lambda b, g: (0, 0), memory_space=pltpu.SMEM),
        ],
        out_shape=[
            jax.ShapeDtypeStruct((1, 1), jnp.float32),
            jax.ShapeDtypeStruct((1, 1), jnp.float32),
        ],
    )(predT, tgtT, wmapT)
    return _LOSS_WEIGHT * loss_sum[0, 0] / cnt[0, 0]


# speedup vs baseline: 9.2823x; 1.0413x over previous
import functools

import jax
import jax.numpy as jnp
from jax.experimental import pallas as pl
from jax.experimental.pallas import tpu as pltpu

_ALPHA = 0.25
_LOSS_WEIGHT = 100.0
_IGNORE_INDEX = 255

_LOG2E = 1.4426950408889634
_LN2 = 0.6931471805599453


def _focal_body(C, Hb, D, W, nsteps, pred_ref, tgt_ref, w_ref, out_ref, acc_ref):
    step = pl.program_id(0) * pl.num_programs(1) + pl.program_id(1)

    @pl.when(step == 0)
    def _init():
        acc_ref[0] = 0.0
        acc_ref[1] = 0.0

    vacc = jnp.zeros((D, W), jnp.float32)
    vcnt = jnp.zeros((D, W), jnp.float32)
    for hh in range(Hb):
        tgt_h = tgt_ref[0, hh]
        acc = jnp.zeros((D, W), jnp.float32)
        xt = jnp.zeros((D, W), jnp.float32)
        for c in range(C):
            x = pred_ref[0, c, hh]
            e = jnp.exp2(jnp.abs(x) * (-_LOG2E))
            t = 1.0 + e
            sp = jnp.maximum(x, 0.0) + jnp.log2(t) * _LN2
            inv = 1.0 / t
            sig = jnp.where(x >= 0.0, inv, e * inv)
            acc = acc + sp * (sig * sig)
            xt = jnp.where(tgt_h == c, x, xt)
        e = jnp.exp2(jnp.abs(xt) * (-_LOG2E))
        t = 1.0 + e
        sp = jnp.maximum(xt, 0.0) + jnp.log2(t) * _LN2
        inv = 1.0 / t
        sig = jnp.where(xt >= 0.0, inv, e * inv)
        oms = 1.0 - sig
        l1 = (sp - xt) * (_ALPHA * (oms * oms))
        l0t = sp * ((1.0 - _ALPHA) * (sig * sig))
        voxel = (1.0 - _ALPHA) * acc + (l1 - l0t)
        vis = tgt_h != _IGNORE_INDEX
        vacc = vacc + voxel * jnp.where(vis, w_ref[hh], 0.0)
        vcnt = vcnt + jnp.where(vis, 1.0, 0.0)

    acc_ref[0] += jnp.sum(vacc)
    acc_ref[1] += jnp.sum(vcnt)

    @pl.when(step == nsteps - 1)
    def _flush():
        out_ref[0, 0] = acc_ref[0]
        out_ref[0, 1] = acc_ref[1]


def kernel(pred, target):
    B, C, H, W, D = pred.shape
    Hb = 20
    nsteps = B * (H // Hb)

    predT = jnp.transpose(pred, (0, 1, 2, 4, 3))
    tgtT = jnp.transpose(target, (0, 1, 3, 2))

    yy = jnp.arange(H, dtype=jnp.float32) - H / 2.0
    xx = jnp.arange(W, dtype=jnp.float32) - W / 2.0
    gy, gx = jnp.meshgrid(yy, xx, indexing="ij")
    cmap = jnp.sqrt(gy * gy + gx * gx)
    cmap = cmap / jnp.maximum(cmap.max(), 1e-12) + 1.0
    wmapT = jnp.broadcast_to(cmap[:, None, :], (H, D, W))

    body = functools.partial(_focal_body, C, Hb, D, W, nsteps)
    out = pl.pallas_call(
        body,
        grid=(B, H // Hb),
        in_specs=[
            pl.BlockSpec((1, C, Hb, D, W), lambda b, g: (b, 0, g, 0, 0)),
            pl.BlockSpec((1, Hb, D, W), lambda b, g: (b, g, 0, 0)),
            pl.BlockSpec((Hb, D, W), lambda b, g: (g, 0, 0)),
        ],
        out_specs=pl.BlockSpec((1, 2), lambda b, g: (0, 0), memory_space=pltpu.SMEM),
        out_shape=jax.ShapeDtypeStruct((1, 2), jnp.float32),
        scratch_shapes=[pltpu.SMEM((2,), jnp.float32)],
    )(predT, tgtT, wmapT)
    return _LOSS_WEIGHT * out[0, 0] / out[0, 1]
